# baseline (speedup 1.0000x reference)
import jax
import jax.numpy as jnp
from jax.experimental import pallas as pl
from jax.experimental.pallas import tpu as pltpu

_NUM_EXPERTS = 16
_TOP_K = 2
_NOISE_STD = 1e-2
_BLOCK_T = 256


def _router_block(x_ref, w_ref, noise_ref, comb_ref, idx_ref, val_ref):
    scores = jnp.dot(x_ref[...], w_ref[...], preferred_element_type=jnp.float32)
    logits = scores + noise_ref[...]
    m = jnp.max(logits, axis=-1, keepdims=True)
    e = jnp.exp(logits - m)
    gates = e / jnp.sum(e, axis=-1, keepdims=True)
    lane = jax.lax.broadcasted_iota(jnp.int32, gates.shape, 1)
    big = jnp.int32(_NUM_EXPERTS)
    v1 = jnp.max(gates, axis=-1, keepdims=True)
    i1 = jnp.min(jnp.where(gates == v1, lane, big), axis=-1, keepdims=True)
    masked = jnp.where(lane == i1, -jnp.inf, gates)
    v2 = jnp.max(masked, axis=-1, keepdims=True)
    i2 = jnp.min(jnp.where(masked == v2, lane, big), axis=-1, keepdims=True)
    denom = v1 + v2 + 1e-9
    g1 = v1 / denom
    g2 = v2 / denom
    comb_ref[...] = jnp.where(lane == i1, g1, jnp.where(lane == i2, g2, 0.0))
    pair = jax.lax.broadcasted_iota(jnp.int32, (i1.shape[0], _TOP_K), 1)
    idx_ref[...] = jnp.where(pair == 0, i1, i2)
    val_ref[...] = jnp.where(pair == 0, g1, g2)


def kernel(x, W):
    n, d = x.shape
    noise = jax.random.normal(
        jax.random.fold_in(jax.random.key(42), 7), (n, _NUM_EXPERTS), jnp.float32
    ) * _NOISE_STD
    grid = n // _BLOCK_T
    comb, idx, val = pl.pallas_call(
        _router_block,
        grid=(grid,),
        in_specs=[
            pl.BlockSpec((_BLOCK_T, d), lambda i: (i, 0)),
            pl.BlockSpec((d, _NUM_EXPERTS), lambda i: (0, 0)),
            pl.BlockSpec((_BLOCK_T, _NUM_EXPERTS), lambda i: (i, 0)),
        ],
        out_specs=[
            pl.BlockSpec((_BLOCK_T, _NUM_EXPERTS), lambda i: (i, 0)),
            pl.BlockSpec((_BLOCK_T, _TOP_K), lambda i: (i, 0)),
            pl.BlockSpec((_BLOCK_T, _TOP_K), lambda i: (i, 0)),
        ],
        out_shape=[
            jax.ShapeDtypeStruct((n, _NUM_EXPERTS), jnp.float32),
            jax.ShapeDtypeStruct((n, _TOP_K), jnp.int32),
            jax.ShapeDtypeStruct((n, _TOP_K), jnp.float32),
        ],
        compiler_params=pltpu.CompilerParams(
            dimension_semantics=("parallel",),
        ),
    )(x, W, noise)
    return comb, idx, val


# speedup vs baseline: 1.2341x; 1.2341x over previous
import jax
import jax.numpy as jnp
from jax.experimental import pallas as pl
from jax.experimental.pallas import tpu as pltpu

_NUM_EXPERTS = 16
_TOP_K = 2
_NOISE_STD = 1e-2
_BLOCK_T = 1024


def _router_block(x_ref, w_ref, noise_ref, comb_ref, idx_ref, val_ref):
    scores = jnp.dot(x_ref[...], w_ref[...], preferred_element_type=jnp.float32)
    logits = scores + noise_ref[...]
    m = jnp.max(logits, axis=-1, keepdims=True)
    e = jnp.exp(logits - m)
    gates = e / jnp.sum(e, axis=-1, keepdims=True)
    lane = jax.lax.broadcasted_iota(jnp.int32, gates.shape, 1)
    big = jnp.int32(_NUM_EXPERTS)
    v1 = jnp.max(gates, axis=-1, keepdims=True)
    i1 = jnp.min(jnp.where(gates == v1, lane, big), axis=-1, keepdims=True)
    masked = jnp.where(lane == i1, -jnp.inf, gates)
    v2 = jnp.max(masked, axis=-1, keepdims=True)
    i2 = jnp.min(jnp.where(masked == v2, lane, big), axis=-1, keepdims=True)
    denom = v1 + v2 + 1e-9
    g1 = v1 / denom
    g2 = v2 / denom
    comb_ref[...] = jnp.where(lane == i1, g1, jnp.where(lane == i2, g2, 0.0))
    pair = jax.lax.broadcasted_iota(jnp.int32, (i1.shape[0], _TOP_K), 1)
    idx_ref[...] = jnp.where(pair == 0, i1, i2)
    val_ref[...] = jnp.where(pair == 0, g1, g2)


def kernel(x, W):
    n, d = x.shape
    noise = jax.random.normal(
        jax.random.fold_in(jax.random.key(42), 7), (n, _NUM_EXPERTS), jnp.float32
    ) * _NOISE_STD
    grid = n // _BLOCK_T
    comb, idx, val = pl.pallas_call(
        _router_block,
        grid=(grid,),
        in_specs=[
            pl.BlockSpec((_BLOCK_T, d), lambda i: (i, 0)),
            pl.BlockSpec((d, _NUM_EXPERTS), lambda i: (0, 0)),
            pl.BlockSpec((_BLOCK_T, _NUM_EXPERTS), lambda i: (i, 0)),
        ],
        out_specs=[
            pl.BlockSpec((_BLOCK_T, _NUM_EXPERTS), lambda i: (i, 0)),
            pl.BlockSpec((_BLOCK_T, _TOP_K), lambda i: (i, 0)),
            pl.BlockSpec((_BLOCK_T, _TOP_K), lambda i: (i, 0)),
        ],
        out_shape=[
            jax.ShapeDtypeStruct((n, _NUM_EXPERTS), jnp.float32),
            jax.ShapeDtypeStruct((n, _TOP_K), jnp.int32),
            jax.ShapeDtypeStruct((n, _TOP_K), jnp.float32),
        ],
        compiler_params=pltpu.CompilerParams(
            dimension_semantics=("parallel",),
        ),
    )(x, W, noise)
    return comb, idx, val
